# Initial kernel scaffold; baseline (speedup 1.0000x reference)
import jax
import jax.numpy as jnp
from jax.experimental import pallas as pl


def kernel(user_nodes, recipe_nodes, features, neigh_idx, W_enc, fc_w, fc_b):
    raise NotImplementedError("write your pallas kernel here")


# speedup vs baseline: 4.4905x; 4.4905x over previous
import functools

import jax
import jax.numpy as jnp
from jax import lax
from jax.experimental import pallas as pl
from jax.experimental.pallas import tpu as pltpu
from jax.experimental.pallas import tpu_sc as plsc

_N = 10000
_D = 128
_K = 32
_B = 8192
_CP = 128
_CV = 16
_NW = 32
_NP = 10240
_NODES_PER_W = _NP // _NW
_CH = 8
_NCH = _NODES_PER_W // _CH
_BPW = _B // _NW
_LANES = 16
_DV = _D // _LANES


def _mesh():
    return plsc.VectorSubcoreMesh(core_axis_name="c", subcore_axis_name="s")


@functools.partial(
    pl.kernel,
    out_type=jax.ShapeDtypeStruct((_NP, _D), jnp.float32),
    mesh=_mesh(),
    scratch_types=[
        pltpu.VMEM((_CH * _K,), jnp.int32),
        pltpu.VMEM((_CH * _K, _D), jnp.float32),
        pltpu.VMEM((_CH, _D), jnp.float32),
        pltpu.SemaphoreType.DMA,
    ],
)
def _neigh_sum(nidx_hbm, feat_hbm, out_hbm, idx_v, rows_v, out_v, sem):
    wid = lax.axis_index("s") * 2 + lax.axis_index("c")
    base = wid * _NODES_PER_W

    def chunk(ci, carry):
        nb = base + ci * _CH
        pltpu.sync_copy(nidx_hbm.at[pl.ds(nb * _K, _CH * _K)], idx_v)
        pltpu.async_copy(feat_hbm.at[idx_v], rows_v, sem).wait()
        for j in range(_CH):
            init = tuple(jnp.zeros((_LANES,), jnp.float32) for _ in range(_DV))

            def body(k, accs, j=j):
                return tuple(
                    accs[d] + rows_v[j * _K + k, pl.ds(d * _LANES, _LANES)]
                    for d in range(_DV)
                )

            accs = lax.fori_loop(0, _K, body, init)
            for d in range(_DV):
                out_v[j, pl.ds(d * _LANES, _LANES)] = accs[d]
        pltpu.sync_copy(out_v, out_hbm.at[pl.ds(nb, _CH)])
        return carry

    lax.fori_loop(0, _NCH, chunk, 0)


def _tc_body(f_ref, ns_ref, w1t_ref, w2t_ref, p1_ref, p2_ref, su_ref, sr_ref):
    z = jnp.dot(f_ref[...], w1t_ref[...], preferred_element_type=jnp.float32)
    z = z + jnp.dot(ns_ref[...] * (1.0 / _K), w2t_ref[...],
                    preferred_element_type=jnp.float32)
    z = jnp.maximum(z, 0.0)
    su_ref[...] = jnp.dot(z, p1_ref[...], preferred_element_type=jnp.float32)
    sr_ref[...] = jnp.dot(z, p2_ref[...], preferred_element_type=jnp.float32)


_RB = 512


def _node_scores(feat_p, ns, w1t, w2t, p1, p2):
    grid = _NP // _RB
    return pl.pallas_call(
        _tc_body,
        grid=(grid,),
        in_specs=[
            pl.BlockSpec((_RB, _D), lambda i: (i, 0)),
            pl.BlockSpec((_RB, _D), lambda i: (i, 0)),
            pl.BlockSpec((_D, _D), lambda i: (0, 0)),
            pl.BlockSpec((_D, _D), lambda i: (0, 0)),
            pl.BlockSpec((_D, _CP), lambda i: (0, 0)),
            pl.BlockSpec((_D, _CP), lambda i: (0, 0)),
        ],
        out_specs=[
            pl.BlockSpec((_RB, _CP), lambda i: (i, 0)),
            pl.BlockSpec((_RB, _CP), lambda i: (i, 0)),
        ],
        out_shape=[
            jax.ShapeDtypeStruct((_NP, _CP), jnp.float32),
            jax.ShapeDtypeStruct((_NP, _CP), jnp.float32),
        ],
    )(feat_p, ns, w1t, w2t, p1, p2)


@functools.partial(
    pl.kernel,
    out_type=jax.ShapeDtypeStruct((_B, _CV), jnp.float32),
    mesh=_mesh(),
    scratch_types=[
        pltpu.VMEM((_BPW,), jnp.int32),
        pltpu.VMEM((_BPW,), jnp.int32),
        pltpu.VMEM((_BPW, _CP), jnp.float32),
        pltpu.VMEM((_BPW, _CP), jnp.float32),
        pltpu.VMEM((_BPW, _CV), jnp.float32),
        pltpu.VMEM((_CV,), jnp.float32),
        pltpu.SemaphoreType.DMA,
        pltpu.SemaphoreType.DMA,
    ],
)
def _pair_scores(su_hbm, sr_hbm, u_hbm, r_hbm, b_hbm, out_hbm,
                 ui_v, ri_v, su_v, sr_v, o_v, b_v, sem_u, sem_r):
    wid = lax.axis_index("s") * 2 + lax.axis_index("c")
    base = wid * _BPW
    pltpu.sync_copy(b_hbm, b_v)
    pltpu.sync_copy(u_hbm.at[pl.ds(base, _BPW)], ui_v)
    pltpu.sync_copy(r_hbm.at[pl.ds(base, _BPW)], ri_v)
    cu = pltpu.async_copy(su_hbm.at[ui_v], su_v, sem_u)
    cr = pltpu.async_copy(sr_hbm.at[ri_v], sr_v, sem_r)
    cu.wait()
    cr.wait()
    bias = b_v[...]

    def body(r, carry):
        o_v[r, :] = (su_v[r, pl.ds(0, _CV)] + sr_v[r, pl.ds(0, _CV)] + bias)
        return carry

    lax.fori_loop(0, _BPW, body, 0)
    pltpu.sync_copy(o_v, out_hbm.at[pl.ds(base, _BPW)])


def kernel(user_nodes, recipe_nodes, features, neigh_idx, W_enc, fc_w, fc_b):
    feat_p = jnp.pad(features, ((0, _NP - _N), (0, 0)))
    nidx_flat = jnp.pad(neigh_idx.astype(jnp.int32),
                        ((0, _NP - _N), (0, 0))).reshape(-1)
    ns = _neigh_sum(nidx_flat, feat_p)

    w1t = W_enc[:, :_D].T
    w2t = W_enc[:, _D:].T
    p1 = jnp.pad(fc_w[:, :_D], ((0, _CP - fc_w.shape[0]), (0, 0))).T
    p2 = jnp.pad(fc_w[:, _D:], ((0, _CP - fc_w.shape[0]), (0, 0))).T
    su, sr = _node_scores(feat_p, ns, w1t, w2t, p1, p2)

    bias_p = jnp.pad(fc_b, (0, _CV - fc_b.shape[0]))
    out = _pair_scores(su, sr, user_nodes.astype(jnp.int32),
                       recipe_nodes.astype(jnp.int32), bias_p)
    return out[:, :fc_b.shape[0]]


# speedup vs baseline: 5.1175x; 1.1396x over previous
import functools

import jax
import jax.numpy as jnp
from jax import lax
from jax.experimental import pallas as pl
from jax.experimental.pallas import tpu as pltpu
from jax.experimental.pallas import tpu_sc as plsc

_N = 10000
_D = 128
_K = 32
_B = 8192
_CP = 128
_CV = 16
_NW = 32
_NP = 10240
_NODES_PER_W = _NP // _NW
_CH = 8
_NCH = _NODES_PER_W // _CH
_BPW = _B // _NW
_LANES = 16
_DV = _D // _LANES


def _mesh():
    return plsc.VectorSubcoreMesh(core_axis_name="c", subcore_axis_name="s")


@functools.partial(
    pl.kernel,
    out_type=jax.ShapeDtypeStruct((_NP, _D), jnp.float32),
    mesh=_mesh(),
    scratch_types=[
        pltpu.VMEM((_CH * _K,), jnp.int32),
        pltpu.VMEM((_CH * _K,), jnp.int32),
        pltpu.VMEM((_CH * _K, _D), jnp.float32),
        pltpu.VMEM((_CH * _K, _D), jnp.float32),
        pltpu.VMEM((_CH, _D), jnp.float32),
        pltpu.SemaphoreType.DMA,
        pltpu.SemaphoreType.DMA,
    ],
)
def _neigh_sum(nidx_hbm, feat_hbm, out_hbm,
               idx_v0, idx_v1, rows_v0, rows_v1, out_v, sem0, sem1):
    wid = lax.axis_index("s") * 2 + lax.axis_index("c")
    base = wid * _NODES_PER_W
    idx_v = (idx_v0, idx_v1)
    rows_v = (rows_v0, rows_v1)
    sem = (sem0, sem1)

    pltpu.sync_copy(nidx_hbm.at[pl.ds(base * _K, _CH * _K)], idx_v[0])
    pltpu.async_copy(feat_hbm.at[idx_v[0]], rows_v[0], sem[0])

    def outer(co, carry):
        for b in range(2):
            ci = co * 2 + b
            nb = base + ci * _CH

            @pl.when(ci + 1 < _NCH)
            def _issue(b=b, ci=ci):
                pltpu.sync_copy(
                    nidx_hbm.at[pl.ds((base + (ci + 1) * _CH) * _K, _CH * _K)],
                    idx_v[1 - b])
                pltpu.async_copy(feat_hbm.at[idx_v[1 - b]], rows_v[1 - b],
                                 sem[1 - b])

            pltpu.make_async_copy(feat_hbm.at[idx_v[b]], rows_v[b],
                                  sem[b]).wait()

            rv = rows_v[b]
            for j in range(_CH):
                init = tuple(
                    rv[j * _K, pl.ds(d * _LANES, _LANES)] for d in range(_DV))

                def kbody(kk, accs, j=j, rv=rv):
                    row = j * _K + 1 + kk * 4
                    for r in range(4):
                        accs = tuple(
                            accs[d] + rv[row + r, pl.ds(d * _LANES, _LANES)]
                            for d in range(_DV))
                    return accs

                accs = lax.fori_loop(0, 7, kbody, init)
                row0 = j * _K + 29
                for r in range(3):
                    accs = tuple(
                        accs[d] + rv[row0 + r, pl.ds(d * _LANES, _LANES)]
                        for d in range(_DV))
                for d in range(_DV):
                    out_v[j, pl.ds(d * _LANES, _LANES)] = accs[d]
            pltpu.sync_copy(out_v, out_hbm.at[pl.ds(nb, _CH)])
        return carry

    lax.fori_loop(0, _NCH // 2, outer, 0)


def _tc_body(f_ref, ns_ref, w1t_ref, w2t_ref, p1_ref, p2_ref, su_ref, sr_ref):
    z = jnp.dot(f_ref[...], w1t_ref[...], preferred_element_type=jnp.float32)
    z = z + jnp.dot(ns_ref[...] * (1.0 / _K), w2t_ref[...],
                    preferred_element_type=jnp.float32)
    z = jnp.maximum(z, 0.0)
    su_ref[...] = jnp.dot(z, p1_ref[...], preferred_element_type=jnp.float32)
    sr_ref[...] = jnp.dot(z, p2_ref[...], preferred_element_type=jnp.float32)


_RB = 512


def _node_scores(feat_p, ns, w1t, w2t, p1, p2):
    grid = _NP // _RB
    return pl.pallas_call(
        _tc_body,
        grid=(grid,),
        in_specs=[
            pl.BlockSpec((_RB, _D), lambda i: (i, 0)),
            pl.BlockSpec((_RB, _D), lambda i: (i, 0)),
            pl.BlockSpec((_D, _D), lambda i: (0, 0)),
            pl.BlockSpec((_D, _D), lambda i: (0, 0)),
            pl.BlockSpec((_D, _CP), lambda i: (0, 0)),
            pl.BlockSpec((_D, _CP), lambda i: (0, 0)),
        ],
        out_specs=[
            pl.BlockSpec((_RB, _CP), lambda i: (i, 0)),
            pl.BlockSpec((_RB, _CP), lambda i: (i, 0)),
        ],
        out_shape=[
            jax.ShapeDtypeStruct((_NP, _CP), jnp.float32),
            jax.ShapeDtypeStruct((_NP, _CP), jnp.float32),
        ],
    )(feat_p, ns, w1t, w2t, p1, p2)


@functools.partial(
    pl.kernel,
    out_type=jax.ShapeDtypeStruct((_B, _CV), jnp.float32),
    mesh=_mesh(),
    scratch_types=[
        pltpu.VMEM((_BPW,), jnp.int32),
        pltpu.VMEM((_BPW,), jnp.int32),
        pltpu.VMEM((_BPW, _CP), jnp.float32),
        pltpu.VMEM((_BPW, _CP), jnp.float32),
        pltpu.VMEM((_BPW, _CV), jnp.float32),
        pltpu.VMEM((_CV,), jnp.float32),
        pltpu.SemaphoreType.DMA,
        pltpu.SemaphoreType.DMA,
    ],
)
def _pair_scores(su_hbm, sr_hbm, u_hbm, r_hbm, b_hbm, out_hbm,
                 ui_v, ri_v, su_v, sr_v, o_v, b_v, sem_u, sem_r):
    wid = lax.axis_index("s") * 2 + lax.axis_index("c")
    base = wid * _BPW
    pltpu.sync_copy(b_hbm, b_v)
    pltpu.sync_copy(u_hbm.at[pl.ds(base, _BPW)], ui_v)
    pltpu.sync_copy(r_hbm.at[pl.ds(base, _BPW)], ri_v)
    cu = pltpu.async_copy(su_hbm.at[ui_v], su_v, sem_u)
    cr = pltpu.async_copy(sr_hbm.at[ri_v], sr_v, sem_r)
    cu.wait()
    cr.wait()
    bias = b_v[...]

    def body(r, carry):
        o_v[r, :] = (su_v[r, pl.ds(0, _CV)] + sr_v[r, pl.ds(0, _CV)] + bias)
        return carry

    lax.fori_loop(0, _BPW, body, 0)
    pltpu.sync_copy(o_v, out_hbm.at[pl.ds(base, _BPW)])


def kernel(user_nodes, recipe_nodes, features, neigh_idx, W_enc, fc_w, fc_b):
    feat_p = jnp.pad(features, ((0, _NP - _N), (0, 0)))
    nidx_flat = jnp.pad(neigh_idx.astype(jnp.int32),
                        ((0, _NP - _N), (0, 0))).reshape(-1)
    ns = _neigh_sum(nidx_flat, feat_p)

    w1t = W_enc[:, :_D].T
    w2t = W_enc[:, _D:].T
    p1 = jnp.pad(fc_w[:, :_D], ((0, _CP - fc_w.shape[0]), (0, 0))).T
    p2 = jnp.pad(fc_w[:, _D:], ((0, _CP - fc_w.shape[0]), (0, 0))).T
    su, sr = _node_scores(feat_p, ns, w1t, w2t, p1, p2)

    bias_p = jnp.pad(fc_b, (0, _CV - fc_b.shape[0]))
    out = _pair_scores(su, sr, user_nodes.astype(jnp.int32),
                       recipe_nodes.astype(jnp.int32), bias_p)
    return out[:, :fc_b.shape[0]]
